# baseline (speedup 1.0000x reference)
import functools

import jax
import jax.numpy as jnp
from jax import lax
from jax.experimental import pallas as pl
from jax.experimental.pallas import tpu as pltpu
from jax.experimental.pallas import tpu_sc as plsc


def _panel_kernel(row_ref, col_ref, out_ref, *, h, w, d):
    hw = h * w
    p = jax.lax.broadcasted_iota(jnp.int32, (hw, max(h, w)), 0)
    q = jax.lax.broadcasted_iota(jnp.int32, (hw, max(h, w)), 1)
    sel_col = (p % w == q).astype(jnp.float32)[:, :w]
    sel_row = (p // w == q).astype(jnp.float32)[:, :h]
    out_ref[:, :d] = jax.lax.dot_general(
        sel_col, col_ref[:w, :], (((1,), (0,)), ((), ())),
        preferred_element_type=jnp.float32,
        precision=jax.lax.Precision.HIGHEST)
    out_ref[:, d:] = jax.lax.dot_general(
        sel_row, row_ref[:h, :], (((1,), (0,)), ((), ())),
        preferred_element_type=jnp.float32,
        precision=jax.lax.Precision.HIGHEST)


_NTILE = 28
_PPT = 7


def _sc_body(panel_ref, out_ref, idx_ref, rows_ref, sem, *, b):
    wid = lax.axis_index("s") * 2 + lax.axis_index("c")

    @pl.when(wid < _NTILE)
    def _():
        p0 = wid * _PPT
        for k in range(_PPT):
            for n in range(b // 16):
                idx_ref[pl.ds(b * k + 16 * n, 16)] = jnp.full(
                    (16,), k, jnp.int32) + p0
        for k in range(_PPT):
            pltpu.make_async_copy(
                panel_ref.at[idx_ref.at[pl.ds(b * k, b)]],
                rows_ref.at[k], sem).start()
        for k in range(_PPT):
            pltpu.make_async_copy(
                panel_ref.at[idx_ref.at[pl.ds(b * k, b)]],
                rows_ref.at[k], sem).wait()
        pltpu.sync_copy(rows_ref, out_ref.at[pl.ds(p0, _PPT)])


def kernel(x, row_embed, col_embed):
    b = x.shape[0]
    h, w = x.shape[2], x.shape[3]
    n, d = row_embed.shape
    hw = h * w
    panel = pl.pallas_call(
        functools.partial(_panel_kernel, h=h, w=w, d=d),
        in_specs=[
            pl.BlockSpec((n, d), lambda: (0, 0)),
            pl.BlockSpec((n, d), lambda: (0, 0)),
        ],
        out_specs=pl.BlockSpec((hw, 2 * d), lambda: (0, 0)),
        out_shape=jax.ShapeDtypeStruct((hw, 2 * d), jnp.float32),
    )(row_embed, col_embed)

    mesh = plsc.VectorSubcoreMesh(core_axis_name="c", subcore_axis_name="s")
    sc = pl.kernel(
        functools.partial(_sc_body, b=b),
        mesh=mesh,
        out_type=jax.ShapeDtypeStruct((hw, b, 2 * d), jnp.float32),
        scratch_types=[
            pltpu.VMEM((_PPT * b,), jnp.int32),
            pltpu.VMEM((_PPT, b, 2 * d), jnp.float32),
            pltpu.SemaphoreType.DMA,
        ],
    )
    out = sc(panel)
    return jnp.transpose(out.reshape(h, w, b, 2 * d), (2, 3, 0, 1))


# speedup vs baseline: 8.6923x; 8.6923x over previous
import functools

import jax
import jax.numpy as jnp
from jax.experimental import pallas as pl
from jax.experimental.pallas import tpu as pltpu

_CHUNK = 14
_NCHUNK = 14


def _pos_kernel(row_ref, col_ref, out_ref, *stages_and_sems, h, w, d):
    stages = stages_and_sems[:_NCHUNK]
    sems = stages_and_sems[_NCHUNK]
    b = out_ref.shape[1]
    for c in range(_NCHUNK):
        base = c * _CHUNK
        p = base + jax.lax.broadcasted_iota(
            jnp.int32, (_CHUNK, max(h, w)), 0)
        q = jax.lax.broadcasted_iota(jnp.int32, (_CHUNK, max(h, w)), 1)
        sel_col = (p % w == q).astype(jnp.float32)[:, :w]
        sel_row = (p // w == q).astype(jnp.float32)[:, :h]
        top = jax.lax.dot_general(
            sel_col, col_ref[:w, :], (((1,), (0,)), ((), ())),
            preferred_element_type=jnp.float32,
            precision=jax.lax.Precision.HIGHEST)
        bottom = jax.lax.dot_general(
            sel_row, row_ref[:h, :], (((1,), (0,)), ((), ())),
            preferred_element_type=jnp.float32,
            precision=jax.lax.Precision.HIGHEST)
        vec = jnp.concatenate([top, bottom], axis=1)
        stages[c][...] = jnp.broadcast_to(
            vec[:, None, :], (_CHUNK, b, 2 * d))
        pltpu.make_async_copy(
            stages[c], out_ref.at[pl.ds(base, _CHUNK)], sems.at[c]).start()
    for c in range(_NCHUNK):
        pltpu.make_async_copy(
            stages[c], out_ref.at[pl.ds(c * _CHUNK, _CHUNK)],
            sems.at[c]).wait()


def kernel(x, row_embed, col_embed):
    b = x.shape[0]
    h, w = x.shape[2], x.shape[3]
    n, d = row_embed.shape
    hw = h * w
    body = functools.partial(_pos_kernel, h=h, w=w, d=d)
    out = pl.pallas_call(
        body,
        in_specs=[
            pl.BlockSpec((n, d), lambda: (0, 0)),
            pl.BlockSpec((n, d), lambda: (0, 0)),
        ],
        out_specs=pl.BlockSpec(memory_space=pltpu.MemorySpace.HBM),
        out_shape=jax.ShapeDtypeStruct((hw, b, 2 * d), jnp.float32),
        scratch_shapes=(
            [pltpu.VMEM((_CHUNK, b, 2 * d), jnp.float32)
             for _ in range(_NCHUNK)]
            + [pltpu.SemaphoreType.DMA((_NCHUNK,))]
        ),
    )(row_embed, col_embed)
    return jnp.transpose(out.reshape(h, w, b, 2 * d), (2, 3, 0, 1))
